# baseline (speedup 1.0000x reference)
import jax
import jax.numpy as jnp
from jax.experimental import pallas as pl
from jax.experimental.pallas import tpu as pltpu

B, IN_DIM, T = 64, 64, 200
FEAT_DIM = 64
CLASSES = 100000
N = B * T
CB = 2000
NBLK = CLASSES // CB


def _a0_frontend(x_ref, wf_ref, feataug_ref, pooled_ref, sumsq_ref):
    U = jax.lax.dot_general(x_ref[...], wf_ref[...],
                            (((1,), (1,)), ((), ())),
                            preferred_element_type=jnp.float32)
    b_idx = jax.lax.broadcasted_iota(jnp.int32, (B, N), 0)
    i_idx = jax.lax.broadcasted_iota(jnp.int32, (B, N), 1)
    P = jnp.where(i_idx // T == b_idx, jnp.float32(1.0 / T), jnp.float32(0.0))
    pooled_ref[...] = jax.lax.dot_general(P, U, (((1,), (0,)), ((), ())),
                                          preferred_element_type=jnp.float32)
    nrm = jnp.maximum(jnp.sqrt(jnp.sum(U * U, axis=1, keepdims=True)), 1e-12)
    feat = U / nrm
    sumsq_ref[...] = jnp.sum(feat * feat)[None, None]
    ones = jnp.ones((N, 1), jnp.float32)
    zeros = jnp.zeros((N, 127 - FEAT_DIM), jnp.float32)
    feataug_ref[...] = jnp.concatenate([feat, ones, zeros], axis=1)


def _b1_segment_sum(labels_ref, feataug_ref, saug_ref):
    saug_ref[...] = jnp.zeros_like(saug_ref)

    def body(i, carry):
        c = labels_ref[i]
        row = feataug_ref[pl.ds(i, 1), :]
        cur = saug_ref[pl.ds(c, 1), :]
        saug_ref[pl.ds(c, 1), :] = cur + row
        return carry

    jax.lax.fori_loop(0, N, body, 0)


def _a1_cls(wc_ref, pooled_ref, out_ref):
    out_ref[...] = jax.lax.dot_general(wc_ref[...], pooled_ref[...],
                                       (((1,), (1,)), ((), ())),
                                       preferred_element_type=jnp.float32)


def _b2_difference(saug_ref, cen_ref, diff_ref, loss_ref):
    jc = pl.program_id(0)
    S = saug_ref[:, :FEAT_DIM]
    n = saug_ref[:, FEAT_DIM:FEAT_DIM + 1]
    cen = cen_ref[...]
    nf = jnp.maximum(n, 1.0)
    diff_ref[...] = (n * cen - S) / nf
    part = jnp.sum(n * cen * cen - 2.0 * cen * S, axis=0, keepdims=True)

    @pl.when(jc == 0)
    def _():
        loss_ref[...] = part

    @pl.when(jc != 0)
    def _():
        loss_ref[...] += part


def kernel(data, labels, centers, W_feat, W_cls):
    X = data.transpose(0, 2, 1).reshape(N, IN_DIM)
    labels_flat = labels.reshape(-1).astype(jnp.int32)

    feataug, pooled, sumsq = pl.pallas_call(
        _a0_frontend,
        out_shape=(
            jax.ShapeDtypeStruct((N, 128), jnp.float32),
            jax.ShapeDtypeStruct((B, FEAT_DIM), jnp.float32),
            jax.ShapeDtypeStruct((1, 1), jnp.float32),
        ),
    )(X, W_feat)

    saug = pl.pallas_call(
        _b1_segment_sum,
        in_specs=[
            pl.BlockSpec(memory_space=pltpu.SMEM),
            pl.BlockSpec(memory_space=pltpu.VMEM),
        ],
        out_specs=pl.BlockSpec(memory_space=pltpu.VMEM),
        out_shape=jax.ShapeDtypeStruct((CLASSES, 128), jnp.float32),
    )(labels_flat, feataug)

    clsT = pl.pallas_call(
        _a1_cls,
        grid=(NBLK,),
        in_specs=[
            pl.BlockSpec((CB, FEAT_DIM), lambda j: (j, 0)),
            pl.BlockSpec((B, FEAT_DIM), lambda j: (0, 0)),
        ],
        out_specs=pl.BlockSpec((CB, B), lambda j: (j, 0)),
        out_shape=jax.ShapeDtypeStruct((CLASSES, B), jnp.float32),
    )(W_cls, pooled)

    difference, lossvec = pl.pallas_call(
        _b2_difference,
        grid=(NBLK,),
        in_specs=[
            pl.BlockSpec((CB, 128), lambda j: (j, 0)),
            pl.BlockSpec((CB, FEAT_DIM), lambda j: (j, 0)),
        ],
        out_specs=(
            pl.BlockSpec((CB, FEAT_DIM), lambda j: (j, 0)),
            pl.BlockSpec((1, FEAT_DIM), lambda j: (0, 0)),
        ),
        out_shape=(
            jax.ShapeDtypeStruct((CLASSES, FEAT_DIM), jnp.float32),
            jax.ShapeDtypeStruct((1, FEAT_DIM), jnp.float32),
        ),
    )(saug, centers)

    center_loss = (sumsq[0, 0] + jnp.sum(lossvec)) / (N * FEAT_DIM)
    cls = clsT.T
    original_labels = labels.astype(jnp.int64)
    return (center_loss, difference, cls, original_labels)
